# baseline (speedup 1.0000x reference)
import functools

import jax
import jax.numpy as jnp
from jax import lax
from jax.experimental import pallas as pl
from jax.experimental.pallas import tpu as pltpu
from jax.experimental.pallas import tpu_sc as plsc

N = 10000
E = 320000
H = 128
TF = 8
ED = 4
NH = 4
HD = H // NH
L = 2

ROW_BLK = 1024

NC = 2
NS = 16
NW = NC * NS
NPAD = 10240
NP4 = NPAD * NH
EP = 327680
EPW = EP // NW

CW1 = 128
NR1 = EP // CW1
RPW1 = NR1 // NW
SCH1 = 8
NSC1 = RPW1 // SCH1

CW2 = 64
NR2 = EP // CW2
RPW2 = NR2 // NW
SCH2 = 32
NSC2 = RPW2 // SCH2

CWA = 64
RING = 2

RRED = NP4 // NW
ROWS_PT = NPAD // NS
ROWS_PW = NPAD // NW
XB = 64

_SC_PARAMS = dict(needs_layout_passes=False)


def _grid_rows(n, blk):
    return (n + blk - 1) // blk


def _inproj_body(thp_ref, x_ref, wtop_ref, wbot_ref, b_ref, o_ref):
    acc = jnp.dot(thp_ref[...], wtop_ref[...], preferred_element_type=jnp.float32)
    acc += jnp.dot(x_ref[...], wbot_ref[...], preferred_element_type=jnp.float32)
    o_ref[...] = acc + b_ref[...]


def _inproj(thp, x, Win, b_in):
    grid = (_grid_rows(N, ROW_BLK),)
    return pl.pallas_call(
        _inproj_body,
        grid=grid,
        in_specs=[
            pl.BlockSpec((ROW_BLK, H), lambda i: (i, 0)),
            pl.BlockSpec((ROW_BLK, TF), lambda i: (i, 0)),
            pl.BlockSpec((H, H), lambda i: (0, 0)),
            pl.BlockSpec((TF, H), lambda i: (0, 0)),
            pl.BlockSpec((1, H), lambda i: (0, 0)),
        ],
        out_specs=pl.BlockSpec((ROW_BLK, H), lambda i: (i, 0)),
        out_shape=jax.ShapeDtypeStruct((N, H), jnp.float32),
    )(thp, x, Win[:H], Win[H:], b_in.reshape(1, H))


def _qkv_body(h_ref, wq_ref, wk_ref, wv_ref, q_ref, k_ref, v_ref):
    h = h_ref[...]
    q_ref[...] = jnp.dot(h, wq_ref[...], preferred_element_type=jnp.float32)
    k_ref[...] = jnp.dot(h, wk_ref[...], preferred_element_type=jnp.float32)
    v_ref[...] = jnp.dot(h, wv_ref[...], preferred_element_type=jnp.float32)


def _qkv(h, Wq_scaled, Wk, Wv):
    grid = (_grid_rows(N, ROW_BLK),)
    hs = jax.ShapeDtypeStruct((N, H), jnp.float32)
    return pl.pallas_call(
        _qkv_body,
        grid=grid,
        in_specs=[
            pl.BlockSpec((ROW_BLK, H), lambda i: (i, 0)),
            pl.BlockSpec((H, H), lambda i: (0, 0)),
            pl.BlockSpec((H, H), lambda i: (0, 0)),
            pl.BlockSpec((H, H), lambda i: (0, 0)),
        ],
        out_specs=[
            pl.BlockSpec((ROW_BLK, H), lambda i: (i, 0)),
            pl.BlockSpec((ROW_BLK, H), lambda i: (i, 0)),
            pl.BlockSpec((ROW_BLK, H), lambda i: (i, 0)),
        ],
        out_shape=[hs, hs, hs],
    )(h, Wq_scaled, Wk, Wv)


def _ebias_body(ea_ref, w_ref, b_ref, o_ref):
    o_ref[...] = (
        jnp.dot(ea_ref[...], w_ref[...], preferred_element_type=jnp.float32)
        + b_ref[...]
    )


def _ebias(edge_attr, We_i, be_i):
    ea2 = edge_attr.reshape(E // 32, 128)
    eye32 = jnp.eye(32, dtype=jnp.float32)
    wtil = jnp.einsum("ab,dj->adbj", eye32, We_i).reshape(128, 128)
    btil = jnp.tile(be_i, 32).reshape(1, 128)
    rows = E // 32
    grid = (_grid_rows(rows, ROW_BLK),)
    out = pl.pallas_call(
        _ebias_body,
        grid=grid,
        in_specs=[
            pl.BlockSpec((ROW_BLK, 128), lambda i: (i, 0)),
            pl.BlockSpec((128, 128), lambda i: (0, 0)),
            pl.BlockSpec((1, 128), lambda i: (0, 0)),
        ],
        out_specs=pl.BlockSpec((ROW_BLK, 128), lambda i: (i, 0)),
        out_shape=jax.ShapeDtypeStruct((rows, 128), jnp.float32),
    )(ea2, wtil, btil)
    return out.reshape(E, NH)


def _post_body(h_ref, a0_ref, a1_ref, ix_ref, wo_ref, bo_ref, lg_ref,
               lb_ref, w1_ref, b1_ref, w2_ref, b2_ref, fg_ref, fb_ref,
               o_ref, *, final_ln):
    h = h_ref[...]
    aggr = (a0_ref[...] + a1_ref[...]) * ix_ref[...]
    hm = jnp.dot(aggr, wo_ref[...], preferred_element_type=jnp.float32) + bo_ref[...]
    t = h + hm
    mu = jnp.mean(t, axis=-1, keepdims=True)
    var = jnp.mean((t - mu) ** 2, axis=-1, keepdims=True)
    h1 = (t - mu) * lax.rsqrt(var + 1e-5) * lg_ref[...] + lb_ref[...]
    f = jnp.maximum(
        jnp.dot(h1, w1_ref[...], preferred_element_type=jnp.float32) + b1_ref[...],
        0.0,
    )
    f = jnp.dot(f, w2_ref[...], preferred_element_type=jnp.float32) + b2_ref[...]
    o = h1 + f
    if final_ln:
        mu2 = jnp.mean(o, axis=-1, keepdims=True)
        var2 = jnp.mean((o - mu2) ** 2, axis=-1, keepdims=True)
        o = (o - mu2) * lax.rsqrt(var2 + 1e-5) * fg_ref[...] + fb_ref[...]
    o_ref[...] = o


def _post(h, a0, a1, invx, Wo_i, bo_i, lg_i, lb_i, W1_i, b1_i, W2_i, b2_i,
          fn_g, fn_b, final_ln):
    grid = (_grid_rows(N, ROW_BLK),)
    return pl.pallas_call(
        functools.partial(_post_body, final_ln=final_ln),
        grid=grid,
        in_specs=[
            pl.BlockSpec((ROW_BLK, H), lambda i: (i, 0)),
            pl.BlockSpec((ROW_BLK, H), lambda i: (i, 0)),
            pl.BlockSpec((ROW_BLK, H), lambda i: (i, 0)),
            pl.BlockSpec((ROW_BLK, H), lambda i: (i, 0)),
            pl.BlockSpec((H, H), lambda i: (0, 0)),
            pl.BlockSpec((1, H), lambda i: (0, 0)),
            pl.BlockSpec((1, H), lambda i: (0, 0)),
            pl.BlockSpec((1, H), lambda i: (0, 0)),
            pl.BlockSpec((H, 2 * H), lambda i: (0, 0)),
            pl.BlockSpec((1, 2 * H), lambda i: (0, 0)),
            pl.BlockSpec((2 * H, H), lambda i: (0, 0)),
            pl.BlockSpec((1, H), lambda i: (0, 0)),
            pl.BlockSpec((1, H), lambda i: (0, 0)),
            pl.BlockSpec((1, H), lambda i: (0, 0)),
        ],
        out_specs=pl.BlockSpec((ROW_BLK, H), lambda i: (i, 0)),
        out_shape=jax.ShapeDtypeStruct((N, H), jnp.float32),
    )(h, a0, a1, invx, Wo_i, bo_i.reshape(1, H), lg_i.reshape(1, H),
      lb_i.reshape(1, H), W1_i, b1_i.reshape(1, 2 * H), W2_i,
      b2_i.reshape(1, H), fn_g.reshape(1, H), fn_b.reshape(1, H))


def _sc_mesh():
    return plsc.VectorSubcoreMesh(core_axis_name="c", subcore_axis_name="s")


def _edge_pass1(qhp, kvh, srcf, dstf, ebp):
    @functools.partial(
        pl.kernel,
        mesh=_sc_mesh(),
        compiler_params=pltpu.CompilerParams(**_SC_PARAMS),
        out_type=[
            jax.ShapeDtypeStruct((EP * NH,), jnp.float32),
            jax.ShapeDtypeStruct((NW * NP4,), jnp.float32),
            jax.ShapeDtypeStruct((EP, H), jnp.float32),
        ],
        scratch_types=[
            pltpu.VMEM((SCH1 * CW1,), jnp.int32),
            pltpu.VMEM((SCH1 * CW1,), jnp.int32),
            pltpu.VMEM((SCH1 * CW1 * NH,), jnp.float32),
            pltpu.VMEM((SCH1 * CW1 * NH,), jnp.float32),
            [pltpu.VMEM((CWA, H), jnp.float32)] * RING,
            [pltpu.VMEM((CWA, 2 * H), jnp.float32)] * RING,
            pltpu.VMEM((NP4,), jnp.float32),
            [pltpu.SemaphoreType.DMA] * RING,
            [pltpu.SemaphoreType.DMA] * RING,
        ],
    )
    def kern(qh_h, kh_h, src_h, dst_h, eb_h, ex_h, dp_h, ve_h,
             srcw, dstw, ebw, exw, qbufs, kbufs, dpart, sqs, sks):
        cid = lax.axis_index("c")
        sid = lax.axis_index("s")
        wid = sid * NC + cid
        wbase = wid * RPW1
        zero = jnp.zeros((16,), jnp.float32)
        nsub = SCH1 * CW1 // CWA

        def zbody(i, carry):
            dpart[pl.ds(i * 16, 16)] = zero
            return carry

        lax.fori_loop(0, NP4 // 16, zbody, 0)

        def g_issue(c, lane):
            pltpu.async_copy(
                qh_h.at[dstw.at[pl.ds(c * CWA, CWA)]], qbufs[lane], sqs[lane])
            pltpu.async_copy(
                kh_h.at[srcw.at[pl.ds(c * CWA, CWA)]], kbufs[lane], sks[lane])

        def g_wait(c, lane):
            pltpu.make_async_copy(
                qh_h.at[dstw.at[pl.ds(c * CWA, CWA)]],
                qbufs[lane], sqs[lane]).wait()
            pltpu.make_async_copy(
                kh_h.at[srcw.at[pl.ds(c * CWA, CWA)]],
                kbufs[lane], sks[lane]).wait()

        def do_chunk(c, lane):
            exbase = (c >> 1) * CW1 * NH + (c & 1) * CWA
            qbuf = qbufs[lane]
            kbuf = kbufs[lane]

            def group(g, c2):
                evec = lax.iota(jnp.int32, 16) + g * 16
                accs = [zero, zero, zero, zero]
                for d in range(H):
                    dv = jnp.full((16,), d, jnp.int32)
                    qv = plsc.load_gather(qbuf, [evec, dv])
                    kv = plsc.load_gather(kbuf, [evec, dv])
                    accs[d // HD] = accs[d // HD] + qv * kv
                dstl = dstw[pl.ds(c * CWA + g * 16, 16)]
                for j in range(NH):
                    off = exbase + j * CW1 + g * 16
                    sv = accs[j] + ebw[pl.ds(off, 16)]
                    exv = jnp.exp(sv)
                    exw[pl.ds(off, 16)] = exv
                    plsc.addupdate_scatter(dpart, [dstl * NH + j], exv)
                return c2

            lax.fori_loop(0, CWA // 16, group, 0)

        def superchunk(sc, carry):
            rb = wbase + sc * SCH1
            pltpu.sync_copy(src_h.at[pl.ds(rb * CW1, SCH1 * CW1)], srcw)
            pltpu.sync_copy(dst_h.at[pl.ds(rb * CW1, SCH1 * CW1)], dstw)
            pltpu.sync_copy(
                eb_h.at[pl.ds(rb * CW1 * NH, SCH1 * CW1 * NH)], ebw)
            for lane in range(RING - 1):
                g_issue(lane, lane)

            def quad(q4, c2):
                for lane in range(RING):
                    c = q4 * RING + lane
                    g_wait(c, lane)

                    @pl.when(c + RING - 1 < nsub)
                    def _():
                        g_issue(c + RING - 1, (lane + RING - 1) % RING)

                    do_chunk(c, lane)
                    pltpu.sync_copy(
                        kbufs[lane].at[:, pl.ds(H, H)],
                        ve_h.at[pl.ds(rb * CW1 + c * CWA, CWA), :])
                return c2

            lax.fori_loop(0, nsub // RING, quad, 0)
            pltpu.sync_copy(
                exw, ex_h.at[pl.ds(rb * CW1 * NH, SCH1 * CW1 * NH)])
            return carry

        lax.fori_loop(0, NSC1, superchunk, 0)
        pltpu.sync_copy(dpart, dp_h.at[pl.ds(wid * NP4, NP4)])

    return kern(qhp, kvh, srcf, dstf, ebp)


def _denom_reduce(dparts):
    @functools.partial(
        pl.kernel,
        mesh=_sc_mesh(),
        compiler_params=pltpu.CompilerParams(**_SC_PARAMS),
        out_type=[jax.ShapeDtypeStruct((NP4,), jnp.float32)],
        scratch_types=[
            pltpu.VMEM((RRED,), jnp.float32),
            pltpu.VMEM((RRED,), jnp.float32),
        ],
    )
    def kern(dp_h, invd_h, acc, tmp):
        cid = lax.axis_index("c")
        sid = lax.axis_index("s")
        wid = sid * NC + cid
        zero = jnp.zeros((16,), jnp.float32)

        def zacc(i, carry):
            acc[pl.ds(i * 16, 16)] = zero
            return carry

        lax.fori_loop(0, RRED // 16, zacc, 0)

        def red(p, carry):
            pltpu.sync_copy(dp_h.at[pl.ds(p * NP4 + wid * RRED, RRED)], tmp)

            def addv(i, c2):
                acc[pl.ds(i * 16, 16)] = (
                    acc[pl.ds(i * 16, 16)] + tmp[pl.ds(i * 16, 16)])
                return c2

            lax.fori_loop(0, RRED // 16, addv, 0)
            return carry

        lax.fori_loop(0, NW, red, 0)

        def inv(i, carry):
            acc[pl.ds(i * 16, 16)] = 1.0 / (acc[pl.ds(i * 16, 16)] + 1e-16)
            return carry

        lax.fori_loop(0, RRED // 16, inv, 0)
        pltpu.sync_copy(acc, invd_h.at[pl.ds(wid * RRED, RRED)])

    (out,) = kern(dparts)
    return out


def _edge_pass2a(ve, dst3, exp_, zrows):
    @functools.partial(
        pl.kernel,
        mesh=_sc_mesh(),
        compiler_params=pltpu.CompilerParams(**_SC_PARAMS),
        out_type=[
            jax.ShapeDtypeStruct((NPAD, H), jnp.float32),
            jax.ShapeDtypeStruct((NPAD, H), jnp.float32),
        ],
        scratch_types=[
            pltpu.VMEM((SCH2, 1, CW2), jnp.int32),
            pltpu.VMEM((SCH2 * CW2 * NH,), jnp.float32),
            pltpu.VMEM((CW2, H), jnp.float32),
            pltpu.VMEM((CW2, H), jnp.float32),
            pltpu.VMEM((CW2, H), jnp.float32),
            pltpu.VMEM((CW2, H), jnp.float32),
            pltpu.VMEM_SHARED((NPAD, H), jnp.float32),
            pltpu.SemaphoreType.DMA,
            pltpu.SemaphoreType.DMA,
            pltpu.SemaphoreType.DMA,
            pltpu.SemaphoreType.DMA,
        ],
    )
    def kern(ve_h, dst_h, ex_h, z_h, a0_h, a1_h,
             dstw3, exw, vA, vB, mA, mB, aggr,
             svA, svB, ssA, ssB):
        cid = lax.axis_index("c")
        sid = lax.axis_index("s")
        wid = sid * NC + cid
        wbase = wid * RPW2

        pltpu.sync_copy(z_h, aggr.at[pl.ds(sid * ROWS_PT, ROWS_PT)])
        plsc.subcore_barrier()

        def v_issue(rb, jj, vbuf, sv):
            pltpu.async_copy(
                ve_h.at[pl.ds((rb + jj) * CW2, CW2), :], vbuf, sv)

        def v_wait(rb, jj, vbuf, sv):
            pltpu.make_async_copy(
                ve_h.at[pl.ds((rb + jj) * CW2, CW2), :], vbuf, sv).wait()

        def s_issue(jj, mbuf, ss):
            pltpu.async_copy(mbuf, aggr.at[dstw3.at[jj, 0]], ss, add=True)

        def s_wait(jj, mbuf, ss):
            pltpu.make_async_copy(mbuf, aggr.at[dstw3.at[jj, 0]], ss).wait()

        def do_msg(jj, vbuf, mbuf):
            exbase = (jj >> 1) * CW1 * NH + (jj & 1) * CW2

            @plsc.parallel_loop(0, CW2 // 16, unroll=2)
            def group(g):
                evec = lax.iota(jnp.int32, 16) + g * 16
                for j in range(NH):
                    ev = exw[pl.ds(exbase + j * CW1 + g * 16, 16)]
                    for d in range(j * HD, (j + 1) * HD):
                        dv = jnp.full((16,), d, jnp.int32)
                        vv = plsc.load_gather(vbuf, [evec, dv])
                        plsc.store_scatter(mbuf, [evec, dv], vv * ev)

        def superchunk(sc, carry):
            rb = wbase + sc * SCH2
            pltpu.sync_copy(dst_h.at[pl.ds(rb, SCH2)], dstw3)
            pltpu.sync_copy(
                ex_h.at[pl.ds(rb * CW2 * NH, SCH2 * CW2 * NH)], exw)
            v_issue(rb, 0, vA, svA)

            def pair(p, c2):
                jj = p * 2
                v_wait(rb, jj, vA, svA)
                v_issue(rb, jj + 1, vB, svB)

                @pl.when(p > 0)
                def _():
                    s_wait(jj, mA, ssA)

                do_msg(jj, vA, mA)
                s_issue(jj, mA, ssA)
                v_wait(rb, jj + 1, vB, svB)

                @pl.when(p < SCH2 // 2 - 1)
                def _():
                    v_issue(rb, jj + 2, vA, svA)

                @pl.when(p > 0)
                def _():
                    s_wait(jj + 1, mB, ssB)

                do_msg(jj + 1, vB, mB)
                s_issue(jj + 1, mB, ssB)
                return c2

            lax.fori_loop(0, SCH2 // 2, pair, 0)
            s_wait(SCH2 - 2, mA, ssA)
            s_wait(SCH2 - 1, mB, ssB)
            return carry

        lax.fori_loop(0, NSC2, superchunk, 0)
        plsc.subcore_barrier()

        @pl.when(cid == 0)
        def _():
            pltpu.sync_copy(aggr.at[pl.ds(sid * ROWS_PT, ROWS_PT)],
                            a0_h.at[pl.ds(sid * ROWS_PT, ROWS_PT)])

        @pl.when(cid == 1)
        def _():
            pltpu.sync_copy(aggr.at[pl.ds(sid * ROWS_PT, ROWS_PT)],
                            a1_h.at[pl.ds(sid * ROWS_PT, ROWS_PT)])

    return kern(ve, dst3, exp_, zrows)


def _edge_pass2b(dstf, exp_, invd):
    @functools.partial(
        pl.kernel,
        mesh=_sc_mesh(),
        compiler_params=pltpu.CompilerParams(**_SC_PARAMS),
        out_type=[
            jax.ShapeDtypeStruct((EP * NH,), jnp.float32),
            jax.ShapeDtypeStruct((NPAD, H), jnp.float32),
        ],
        scratch_types=[
            pltpu.VMEM((SCH1 * CW1,), jnp.int32),
            pltpu.VMEM((SCH1 * CW1 * NH,), jnp.float32),
            pltpu.VMEM((SCH1 * CW1 * NH,), jnp.float32),
            pltpu.VMEM((NP4,), jnp.float32),
            pltpu.VMEM((XB, H), jnp.float32),
        ],
    )
    def kern(dst_h, ex_h, invd_h, al_h, ix_h,
             dstw, exw, albuf, invd_v, xbuf):
        cid = lax.axis_index("c")
        sid = lax.axis_index("s")
        wid = sid * NC + cid
        wbase = wid * RPW1

        pltpu.sync_copy(invd_h, invd_v)

        def xrep(r, carry):
            rbase = wid * ROWS_PW + r * XB

            def xgrp(g, c2):
                nvec = lax.iota(jnp.int32, 16) + (rbase + g * 16)
                lvec = lax.iota(jnp.int32, 16) + g * 16
                for j in range(NH):
                    iv = plsc.load_gather(invd_v, [nvec * NH + j])
                    for d in range(j * HD, (j + 1) * HD):
                        dv = jnp.full((16,), d, jnp.int32)
                        plsc.store_scatter(xbuf, [lvec, dv], iv)
                return c2

            lax.fori_loop(0, XB // 16, xgrp, 0)
            pltpu.sync_copy(xbuf, ix_h.at[pl.ds(rbase, XB)])
            return carry

        lax.fori_loop(0, ROWS_PW // XB, xrep, 0)

        def superchunk(sc, carry):
            rb = wbase + sc * SCH1
            pltpu.sync_copy(dst_h.at[pl.ds(rb * CW1, SCH1 * CW1)], dstw)
            pltpu.sync_copy(
                ex_h.at[pl.ds(rb * CW1 * NH, SCH1 * CW1 * NH)], exw)

            def sub(jj, c2):
                @plsc.parallel_loop(0, CW1 // 16, unroll=2)
                def group(g):
                    dstl = dstw[pl.ds(jj * CW1 + g * 16, 16)]
                    for j in range(NH):
                        iv = plsc.load_gather(invd_v, [dstl * NH + j])
                        off = jj * CW1 * NH + j * CW1 + g * 16
                        albuf[pl.ds(off, 16)] = exw[pl.ds(off, 16)] * iv
                return c2

            lax.fori_loop(0, SCH1, sub, 0)
            pltpu.sync_copy(
                albuf, al_h.at[pl.ds(rb * CW1 * NH, SCH1 * CW1 * NH)])
            return carry

        lax.fori_loop(0, NSC1, superchunk, 0)

    return kern(dstf, exp_, invd)


def kernel(team_h_pooled, x, edge_index, edge_attr, Win, b_in, Wq, Wk, Wv,
           We, be, attn_scale, Wo, bo, ln_g, ln_b, W1, b1, W2, b2,
           fn_g, fn_b):
    npad_e = EP - E
    srcf = jnp.concatenate(
        [edge_index[0], jnp.zeros((npad_e,), jnp.int32)])
    dstf = jnp.concatenate(
        [edge_index[1], jnp.full((npad_e,), NPAD - 1, jnp.int32)])
    dst3 = dstf.reshape(NR2, 1, CW2)
    zrows = jnp.zeros((ROWS_PT, H), jnp.float32)
    h = _inproj(team_h_pooled, x, Win, b_in)
    alphas = []
    for i in range(L):
        eff_scale = (HD ** 0.5) * jnp.clip(jnp.abs(attn_scale[i]), 0.01, None)
        qh, kh, vh = _qkv(h, Wq[i] / eff_scale, Wk[i], Wv[i])
        qhp = jnp.pad(qh, ((0, NPAD - N), (0, 0)))
        kvh = jnp.concatenate([kh, vh], axis=1)
        eb = jnp.pad(_ebias(edge_attr, We[i], be[i]), ((0, npad_e), (0, 0)))
        ebp = eb.reshape(NR1, CW1, NH).transpose(0, 2, 1).reshape(EP * NH)
        exp_, dparts, ve = _edge_pass1(qhp, kvh, srcf, dstf, ebp)
        invd = _denom_reduce(dparts)
        a0, a1 = _edge_pass2a(ve, dst3, exp_, zrows)
        alp, invx = _edge_pass2b(dstf, exp_, invd)
        h = _post(h, a0[:N], a1[:N], invx[:N], Wo[i], bo[i], ln_g[i],
                  ln_b[i], W1[i], b1[i], W2[i], b2[i], fn_g, fn_b,
                  final_ln=(i == L - 1))
        alphas.append(
            alp.reshape(NR1, NH, CW1).transpose(0, 2, 1).reshape(EP, NH)[:E])
    return (h, alphas[0], alphas[1])


# speedup vs baseline: 1.0219x; 1.0219x over previous
import functools

import jax
import jax.numpy as jnp
from jax import lax
from jax.experimental import pallas as pl
from jax.experimental.pallas import tpu as pltpu
from jax.experimental.pallas import tpu_sc as plsc

N = 10000
E = 320000
H = 128
TF = 8
ED = 4
NH = 4
HD = H // NH
L = 2

ROW_BLK = 1024

NC = 2
NS = 16
NW = NC * NS
NPAD = 10240
NP4 = NPAD * NH
EP = 327680
EPW = EP // NW

CW1 = 128
NR1 = EP // CW1
RPW1 = NR1 // NW
SCH1 = 8
NSC1 = RPW1 // SCH1

CW2 = 64
NR2 = EP // CW2
RPW2 = NR2 // NW
SCH2 = 32
NSC2 = RPW2 // SCH2

CWA = 64
RING = 2

RRED = NP4 // NW
ROWS_PT = NPAD // NS
ROWS_PW = NPAD // NW
XB = 64

_SC_PARAMS = dict(needs_layout_passes=False)


def _grid_rows(n, blk):
    return (n + blk - 1) // blk


def _inproj_body(thp_ref, x_ref, wtop_ref, wbot_ref, b_ref, o_ref):
    acc = jnp.dot(thp_ref[...], wtop_ref[...], preferred_element_type=jnp.float32)
    acc += jnp.dot(x_ref[...], wbot_ref[...], preferred_element_type=jnp.float32)
    o_ref[...] = acc + b_ref[...]


def _inproj(thp, x, Win, b_in):
    grid = (_grid_rows(N, ROW_BLK),)
    return pl.pallas_call(
        _inproj_body,
        grid=grid,
        in_specs=[
            pl.BlockSpec((ROW_BLK, H), lambda i: (i, 0)),
            pl.BlockSpec((ROW_BLK, TF), lambda i: (i, 0)),
            pl.BlockSpec((H, H), lambda i: (0, 0)),
            pl.BlockSpec((TF, H), lambda i: (0, 0)),
            pl.BlockSpec((1, H), lambda i: (0, 0)),
        ],
        out_specs=pl.BlockSpec((ROW_BLK, H), lambda i: (i, 0)),
        out_shape=jax.ShapeDtypeStruct((N, H), jnp.float32),
    )(thp, x, Win[:H], Win[H:], b_in.reshape(1, H))


def _qkv_body(h_ref, wq_ref, wk_ref, wv_ref, q_ref, k_ref, v_ref):
    h = h_ref[...]
    q_ref[...] = jnp.dot(h, wq_ref[...], preferred_element_type=jnp.float32)
    k_ref[...] = jnp.dot(h, wk_ref[...], preferred_element_type=jnp.float32)
    v_ref[...] = jnp.dot(h, wv_ref[...], preferred_element_type=jnp.float32)


def _qkv(h, Wq_scaled, Wk, Wv):
    grid = (_grid_rows(N, ROW_BLK),)
    hs = jax.ShapeDtypeStruct((N, H), jnp.float32)
    return pl.pallas_call(
        _qkv_body,
        grid=grid,
        in_specs=[
            pl.BlockSpec((ROW_BLK, H), lambda i: (i, 0)),
            pl.BlockSpec((H, H), lambda i: (0, 0)),
            pl.BlockSpec((H, H), lambda i: (0, 0)),
            pl.BlockSpec((H, H), lambda i: (0, 0)),
        ],
        out_specs=[
            pl.BlockSpec((ROW_BLK, H), lambda i: (i, 0)),
            pl.BlockSpec((ROW_BLK, H), lambda i: (i, 0)),
            pl.BlockSpec((ROW_BLK, H), lambda i: (i, 0)),
        ],
        out_shape=[hs, hs, hs],
    )(h, Wq_scaled, Wk, Wv)


def _ebias_body(ea_ref, w_ref, b_ref, o_ref):
    o_ref[...] = (
        jnp.dot(ea_ref[...], w_ref[...], preferred_element_type=jnp.float32)
        + b_ref[...]
    )


def _ebias(edge_attr, We_i, be_i):
    ea2 = edge_attr.reshape(E // 32, 128)
    eye32 = jnp.eye(32, dtype=jnp.float32)
    wtil = jnp.einsum("ab,dj->adbj", eye32, We_i).reshape(128, 128)
    btil = jnp.tile(be_i, 32).reshape(1, 128)
    rows = E // 32
    grid = (_grid_rows(rows, ROW_BLK),)
    out = pl.pallas_call(
        _ebias_body,
        grid=grid,
        in_specs=[
            pl.BlockSpec((ROW_BLK, 128), lambda i: (i, 0)),
            pl.BlockSpec((128, 128), lambda i: (0, 0)),
            pl.BlockSpec((1, 128), lambda i: (0, 0)),
        ],
        out_specs=pl.BlockSpec((ROW_BLK, 128), lambda i: (i, 0)),
        out_shape=jax.ShapeDtypeStruct((rows, 128), jnp.float32),
    )(ea2, wtil, btil)
    return out.reshape(E, NH)


def _post_body(h_ref, a0_ref, a1_ref, ix_ref, wo_ref, bo_ref, lg_ref,
               lb_ref, w1_ref, b1_ref, w2_ref, b2_ref, fg_ref, fb_ref,
               o_ref, *, final_ln):
    h = h_ref[...]
    aggr = (a0_ref[...] + a1_ref[...]) * ix_ref[...]
    hm = jnp.dot(aggr, wo_ref[...], preferred_element_type=jnp.float32) + bo_ref[...]
    t = h + hm
    mu = jnp.mean(t, axis=-1, keepdims=True)
    var = jnp.mean((t - mu) ** 2, axis=-1, keepdims=True)
    h1 = (t - mu) * lax.rsqrt(var + 1e-5) * lg_ref[...] + lb_ref[...]
    f = jnp.maximum(
        jnp.dot(h1, w1_ref[...], preferred_element_type=jnp.float32) + b1_ref[...],
        0.0,
    )
    f = jnp.dot(f, w2_ref[...], preferred_element_type=jnp.float32) + b2_ref[...]
    o = h1 + f
    if final_ln:
        mu2 = jnp.mean(o, axis=-1, keepdims=True)
        var2 = jnp.mean((o - mu2) ** 2, axis=-1, keepdims=True)
        o = (o - mu2) * lax.rsqrt(var2 + 1e-5) * fg_ref[...] + fb_ref[...]
    o_ref[...] = o


def _post(h, a0, a1, invx, Wo_i, bo_i, lg_i, lb_i, W1_i, b1_i, W2_i, b2_i,
          fn_g, fn_b, final_ln):
    grid = (_grid_rows(N, ROW_BLK),)
    return pl.pallas_call(
        functools.partial(_post_body, final_ln=final_ln),
        grid=grid,
        in_specs=[
            pl.BlockSpec((ROW_BLK, H), lambda i: (i, 0)),
            pl.BlockSpec((ROW_BLK, H), lambda i: (i, 0)),
            pl.BlockSpec((ROW_BLK, H), lambda i: (i, 0)),
            pl.BlockSpec((ROW_BLK, H), lambda i: (i, 0)),
            pl.BlockSpec((H, H), lambda i: (0, 0)),
            pl.BlockSpec((1, H), lambda i: (0, 0)),
            pl.BlockSpec((1, H), lambda i: (0, 0)),
            pl.BlockSpec((1, H), lambda i: (0, 0)),
            pl.BlockSpec((H, 2 * H), lambda i: (0, 0)),
            pl.BlockSpec((1, 2 * H), lambda i: (0, 0)),
            pl.BlockSpec((2 * H, H), lambda i: (0, 0)),
            pl.BlockSpec((1, H), lambda i: (0, 0)),
            pl.BlockSpec((1, H), lambda i: (0, 0)),
            pl.BlockSpec((1, H), lambda i: (0, 0)),
        ],
        out_specs=pl.BlockSpec((ROW_BLK, H), lambda i: (i, 0)),
        out_shape=jax.ShapeDtypeStruct((N, H), jnp.float32),
    )(h, a0, a1, invx, Wo_i, bo_i.reshape(1, H), lg_i.reshape(1, H),
      lb_i.reshape(1, H), W1_i, b1_i.reshape(1, 2 * H), W2_i,
      b2_i.reshape(1, H), fn_g.reshape(1, H), fn_b.reshape(1, H))


def _sc_mesh():
    return plsc.VectorSubcoreMesh(core_axis_name="c", subcore_axis_name="s")


def _edge_pass1(qhp, kh, srcf, dstf, ebp):
    @functools.partial(
        pl.kernel,
        mesh=_sc_mesh(),
        compiler_params=pltpu.CompilerParams(**_SC_PARAMS),
        out_type=[
            jax.ShapeDtypeStruct((EP * NH,), jnp.float32),
            jax.ShapeDtypeStruct((NW * NP4,), jnp.float32),
        ],
        scratch_types=[
            pltpu.VMEM((SCH1 * CW1,), jnp.int32),
            pltpu.VMEM((SCH1 * CW1,), jnp.int32),
            pltpu.VMEM((SCH1 * CW1 * NH,), jnp.float32),
            pltpu.VMEM((SCH1 * CW1 * NH,), jnp.float32),
            [pltpu.VMEM((CWA, H), jnp.float32)] * RING,
            [pltpu.VMEM((CWA, H), jnp.float32)] * RING,
            pltpu.VMEM((NP4,), jnp.float32),
            [pltpu.SemaphoreType.DMA] * RING,
            [pltpu.SemaphoreType.DMA] * RING,
        ],
    )
    def kern(qh_h, kh_h, src_h, dst_h, eb_h, ex_h, dp_h,
             srcw, dstw, ebw, exw, qbufs, kbufs, dpart, sqs, sks):
        cid = lax.axis_index("c")
        sid = lax.axis_index("s")
        wid = sid * NC + cid
        wbase = wid * RPW1
        zero = jnp.zeros((16,), jnp.float32)
        nsub = SCH1 * CW1 // CWA

        def zbody(i, carry):
            dpart[pl.ds(i * 16, 16)] = zero
            return carry

        lax.fori_loop(0, NP4 // 16, zbody, 0)

        def g_issue(c, lane):
            pltpu.async_copy(
                qh_h.at[dstw.at[pl.ds(c * CWA, CWA)]], qbufs[lane], sqs[lane])
            pltpu.async_copy(
                kh_h.at[srcw.at[pl.ds(c * CWA, CWA)]], kbufs[lane], sks[lane])

        def g_wait(c, lane):
            pltpu.make_async_copy(
                qh_h.at[dstw.at[pl.ds(c * CWA, CWA)]],
                qbufs[lane], sqs[lane]).wait()
            pltpu.make_async_copy(
                kh_h.at[srcw.at[pl.ds(c * CWA, CWA)]],
                kbufs[lane], sks[lane]).wait()

        def do_chunk(c, lane):
            exbase = (c >> 1) * CW1 * NH + (c & 1) * CWA
            qbuf = qbufs[lane]
            kbuf = kbufs[lane]

            def group(g, c2):
                evec = lax.iota(jnp.int32, 16) + g * 16
                accs = [zero, zero, zero, zero]
                for d in range(H):
                    dv = jnp.full((16,), d, jnp.int32)
                    qv = plsc.load_gather(qbuf, [evec, dv])
                    kv = plsc.load_gather(kbuf, [evec, dv])
                    accs[d // HD] = accs[d // HD] + qv * kv
                dstl = dstw[pl.ds(c * CWA + g * 16, 16)]
                for j in range(NH):
                    off = exbase + j * CW1 + g * 16
                    sv = accs[j] + ebw[pl.ds(off, 16)]
                    exv = jnp.exp(sv)
                    exw[pl.ds(off, 16)] = exv
                    plsc.addupdate_scatter(dpart, [dstl * NH + j], exv)
                return c2

            lax.fori_loop(0, CWA // 16, group, 0)

        def superchunk(sc, carry):
            rb = wbase + sc * SCH1
            pltpu.sync_copy(src_h.at[pl.ds(rb * CW1, SCH1 * CW1)], srcw)
            pltpu.sync_copy(dst_h.at[pl.ds(rb * CW1, SCH1 * CW1)], dstw)
            pltpu.sync_copy(
                eb_h.at[pl.ds(rb * CW1 * NH, SCH1 * CW1 * NH)], ebw)
            for lane in range(RING - 1):
                g_issue(lane, lane)

            def quad(q4, c2):
                for lane in range(RING):
                    c = q4 * RING + lane
                    g_wait(c, lane)

                    @pl.when(c + RING - 1 < nsub)
                    def _():
                        g_issue(c + RING - 1, (lane + RING - 1) % RING)

                    do_chunk(c, lane)
                return c2

            lax.fori_loop(0, nsub // RING, quad, 0)
            pltpu.sync_copy(
                exw, ex_h.at[pl.ds(rb * CW1 * NH, SCH1 * CW1 * NH)])
            return carry

        lax.fori_loop(0, NSC1, superchunk, 0)
        pltpu.sync_copy(dpart, dp_h.at[pl.ds(wid * NP4, NP4)])

    return kern(qhp, kh, srcf, dstf, ebp)


def _denom_reduce(dparts):
    @functools.partial(
        pl.kernel,
        mesh=_sc_mesh(),
        compiler_params=pltpu.CompilerParams(**_SC_PARAMS),
        out_type=[jax.ShapeDtypeStruct((NP4,), jnp.float32)],
        scratch_types=[
            pltpu.VMEM((RRED,), jnp.float32),
            pltpu.VMEM((RRED,), jnp.float32),
        ],
    )
    def kern(dp_h, invd_h, acc, tmp):
        cid = lax.axis_index("c")
        sid = lax.axis_index("s")
        wid = sid * NC + cid
        zero = jnp.zeros((16,), jnp.float32)

        def zacc(i, carry):
            acc[pl.ds(i * 16, 16)] = zero
            return carry

        lax.fori_loop(0, RRED // 16, zacc, 0)

        def red(p, carry):
            pltpu.sync_copy(dp_h.at[pl.ds(p * NP4 + wid * RRED, RRED)], tmp)

            def addv(i, c2):
                acc[pl.ds(i * 16, 16)] = (
                    acc[pl.ds(i * 16, 16)] + tmp[pl.ds(i * 16, 16)])
                return c2

            lax.fori_loop(0, RRED // 16, addv, 0)
            return carry

        lax.fori_loop(0, NW, red, 0)

        def inv(i, carry):
            acc[pl.ds(i * 16, 16)] = 1.0 / (acc[pl.ds(i * 16, 16)] + 1e-16)
            return carry

        lax.fori_loop(0, RRED // 16, inv, 0)
        pltpu.sync_copy(acc, invd_h.at[pl.ds(wid * RRED, RRED)])

    (out,) = kern(dparts)
    return out


def _edge_pass2a(vh, srcf, dst3, exp_, zrows):
    @functools.partial(
        pl.kernel,
        mesh=_sc_mesh(),
        compiler_params=pltpu.CompilerParams(**_SC_PARAMS),
        out_type=[
            jax.ShapeDtypeStruct((NPAD, H), jnp.float32),
            jax.ShapeDtypeStruct((NPAD, H), jnp.float32),
        ],
        scratch_types=[
            pltpu.VMEM((SCH2 * CW2,), jnp.int32),
            pltpu.VMEM((SCH2, 1, CW2), jnp.int32),
            pltpu.VMEM((SCH2 * CW2 * NH,), jnp.float32),
            pltpu.VMEM((CW2, H), jnp.float32),
            pltpu.VMEM((CW2, H), jnp.float32),
            pltpu.VMEM((CW2, H), jnp.float32),
            pltpu.VMEM((CW2, H), jnp.float32),
            pltpu.VMEM_SHARED((NPAD, H), jnp.float32),
            pltpu.SemaphoreType.DMA,
            pltpu.SemaphoreType.DMA,
            pltpu.SemaphoreType.DMA,
            pltpu.SemaphoreType.DMA,
        ],
    )
    def kern(vh_h, src_h, dst_h, ex_h, z_h, a0_h, a1_h,
             srcw, dstw3, exw, vA, vB, mA, mB, aggr,
             svA, svB, ssA, ssB):
        cid = lax.axis_index("c")
        sid = lax.axis_index("s")
        wid = sid * NC + cid
        wbase = wid * RPW2

        pltpu.sync_copy(z_h, aggr.at[pl.ds(sid * ROWS_PT, ROWS_PT)])
        plsc.subcore_barrier()

        def v_issue(jj, vbuf, sv):
            pltpu.async_copy(vh_h.at[srcw.at[pl.ds(jj * CW2, CW2)]], vbuf, sv)

        def v_wait(jj, vbuf, sv):
            pltpu.make_async_copy(
                vh_h.at[srcw.at[pl.ds(jj * CW2, CW2)]], vbuf, sv).wait()

        def s_issue(jj, mbuf, ss):
            pltpu.async_copy(mbuf, aggr.at[dstw3.at[jj, 0]], ss, add=True)

        def s_wait(jj, mbuf, ss):
            pltpu.make_async_copy(mbuf, aggr.at[dstw3.at[jj, 0]], ss).wait()

        def do_msg(jj, vbuf, mbuf):
            exbase = (jj >> 1) * CW1 * NH + (jj & 1) * CW2

            @plsc.parallel_loop(0, CW2 // 16, unroll=2)
            def group(g):
                evec = lax.iota(jnp.int32, 16) + g * 16
                for j in range(NH):
                    ev = exw[pl.ds(exbase + j * CW1 + g * 16, 16)]
                    for d in range(j * HD, (j + 1) * HD):
                        dv = jnp.full((16,), d, jnp.int32)
                        vv = plsc.load_gather(vbuf, [evec, dv])
                        plsc.store_scatter(mbuf, [evec, dv], vv * ev)

        def superchunk(sc, carry):
            rb = wbase + sc * SCH2
            pltpu.sync_copy(src_h.at[pl.ds(rb * CW2, SCH2 * CW2)], srcw)
            pltpu.sync_copy(dst_h.at[pl.ds(rb, SCH2)], dstw3)
            pltpu.sync_copy(
                ex_h.at[pl.ds(rb * CW2 * NH, SCH2 * CW2 * NH)], exw)
            v_issue(0, vA, svA)

            def pair(p, c2):
                jj = p * 2
                v_wait(jj, vA, svA)
                v_issue(jj + 1, vB, svB)

                @pl.when(p > 0)
                def _():
                    s_wait(jj, mA, ssA)

                do_msg(jj, vA, mA)
                s_issue(jj, mA, ssA)
                v_wait(jj + 1, vB, svB)

                @pl.when(p < SCH2 // 2 - 1)
                def _():
                    v_issue(jj + 2, vA, svA)

                @pl.when(p > 0)
                def _():
                    s_wait(jj + 1, mB, ssB)

                do_msg(jj + 1, vB, mB)
                s_issue(jj + 1, mB, ssB)
                return c2

            lax.fori_loop(0, SCH2 // 2, pair, 0)
            s_wait(SCH2 - 2, mA, ssA)
            s_wait(SCH2 - 1, mB, ssB)
            return carry

        lax.fori_loop(0, NSC2, superchunk, 0)
        plsc.subcore_barrier()

        @pl.when(cid == 0)
        def _():
            pltpu.sync_copy(aggr.at[pl.ds(sid * ROWS_PT, ROWS_PT)],
                            a0_h.at[pl.ds(sid * ROWS_PT, ROWS_PT)])

        @pl.when(cid == 1)
        def _():
            pltpu.sync_copy(aggr.at[pl.ds(sid * ROWS_PT, ROWS_PT)],
                            a1_h.at[pl.ds(sid * ROWS_PT, ROWS_PT)])

    return kern(vh, srcf, dst3, exp_, zrows)


def _edge_pass2b(dstf, exp_, invd):
    @functools.partial(
        pl.kernel,
        mesh=_sc_mesh(),
        compiler_params=pltpu.CompilerParams(**_SC_PARAMS),
        out_type=[
            jax.ShapeDtypeStruct((EP * NH,), jnp.float32),
            jax.ShapeDtypeStruct((NPAD, H), jnp.float32),
        ],
        scratch_types=[
            pltpu.VMEM((SCH1 * CW1,), jnp.int32),
            pltpu.VMEM((SCH1 * CW1 * NH,), jnp.float32),
            pltpu.VMEM((SCH1 * CW1 * NH,), jnp.float32),
            pltpu.VMEM((NP4,), jnp.float32),
            pltpu.VMEM((XB, H), jnp.float32),
        ],
    )
    def kern(dst_h, ex_h, invd_h, al_h, ix_h,
             dstw, exw, albuf, invd_v, xbuf):
        cid = lax.axis_index("c")
        sid = lax.axis_index("s")
        wid = sid * NC + cid
        wbase = wid * RPW1

        pltpu.sync_copy(invd_h, invd_v)

        def xrep(r, carry):
            rbase = wid * ROWS_PW + r * XB

            def xgrp(g, c2):
                nvec = lax.iota(jnp.int32, 16) + (rbase + g * 16)
                lvec = lax.iota(jnp.int32, 16) + g * 16
                for j in range(NH):
                    iv = plsc.load_gather(invd_v, [nvec * NH + j])
                    for d in range(j * HD, (j + 1) * HD):
                        dv = jnp.full((16,), d, jnp.int32)
                        plsc.store_scatter(xbuf, [lvec, dv], iv)
                return c2

            lax.fori_loop(0, XB // 16, xgrp, 0)
            pltpu.sync_copy(xbuf, ix_h.at[pl.ds(rbase, XB)])
            return carry

        lax.fori_loop(0, ROWS_PW // XB, xrep, 0)

        def superchunk(sc, carry):
            rb = wbase + sc * SCH1
            pltpu.sync_copy(dst_h.at[pl.ds(rb * CW1, SCH1 * CW1)], dstw)
            pltpu.sync_copy(
                ex_h.at[pl.ds(rb * CW1 * NH, SCH1 * CW1 * NH)], exw)

            def sub(jj, c2):
                @plsc.parallel_loop(0, CW1 // 16, unroll=2)
                def group(g):
                    dstl = dstw[pl.ds(jj * CW1 + g * 16, 16)]
                    for j in range(NH):
                        iv = plsc.load_gather(invd_v, [dstl * NH + j])
                        off = jj * CW1 * NH + j * CW1 + g * 16
                        albuf[pl.ds(off, 16)] = exw[pl.ds(off, 16)] * iv
                return c2

            lax.fori_loop(0, SCH1, sub, 0)
            pltpu.sync_copy(
                albuf, al_h.at[pl.ds(rb * CW1 * NH, SCH1 * CW1 * NH)])
            return carry

        lax.fori_loop(0, NSC1, superchunk, 0)

    return kern(dstf, exp_, invd)


def kernel(team_h_pooled, x, edge_index, edge_attr, Win, b_in, Wq, Wk, Wv,
           We, be, attn_scale, Wo, bo, ln_g, ln_b, W1, b1, W2, b2,
           fn_g, fn_b):
    npad_e = EP - E
    srcf = jnp.concatenate(
        [edge_index[0], jnp.zeros((npad_e,), jnp.int32)])
    dstf = jnp.concatenate(
        [edge_index[1], jnp.full((npad_e,), NPAD - 1, jnp.int32)])
    dst3 = dstf.reshape(NR2, 1, CW2)
    zrows = jnp.zeros((ROWS_PT, H), jnp.float32)
    h = _inproj(team_h_pooled, x, Win, b_in)
    alphas = []
    for i in range(L):
        eff_scale = (HD ** 0.5) * jnp.clip(jnp.abs(attn_scale[i]), 0.01, None)
        qh, kh, vh = _qkv(h, Wq[i] / eff_scale, Wk[i], Wv[i])
        qhp = jnp.pad(qh, ((0, NPAD - N), (0, 0)))
        eb = jnp.pad(_ebias(edge_attr, We[i], be[i]), ((0, npad_e), (0, 0)))
        ebp = eb.reshape(NR1, CW1, NH).transpose(0, 2, 1).reshape(EP * NH)
        exp_, dparts = _edge_pass1(qhp, kh, srcf, dstf, ebp)
        invd = _denom_reduce(dparts)
        a0, a1 = _edge_pass2a(vh, srcf, dst3, exp_, zrows)
        alp, invx = _edge_pass2b(dstf, exp_, invd)
        h = _post(h, a0[:N], a1[:N], invx[:N], Wo[i], bo[i], ln_g[i],
                  ln_b[i], W1[i], b1[i], W2[i], b2[i], fn_g, fn_b,
                  final_ln=(i == L - 1))
        alphas.append(
            alp.reshape(NR1, NH, CW1).transpose(0, 2, 1).reshape(EP, NH)[:E])
    return (h, alphas[0], alphas[1])
